# baseline (speedup 1.0000x reference)
import jax
import jax.numpy as jnp
from jax.experimental import pallas as pl
from jax.experimental.pallas import tpu as pltpu

HIDDEN = 128


def _round_up(n, m):
    return ((n + m - 1) // m) * m


def _make_body(seq_len, block_t, n_t, b_blk, unroll=8):
    H = HIDDEN
    rem = seq_len - (n_t - 1) * block_t

    def body(x_ref,
             w_ih0_ref,
             w_step_ref,
             b1_ref,
             w_d1_ref, b_d1_ref,
             w_d2_ref, b_d2_ref,
             w_head_ref, b_head_ref,
             out_ref,
             h0_ref, c0_ref, h1_ref, c1_ref,
             rec0_ref,
             g0_ref):
        t_blk = pl.program_id(1)

        @pl.when(t_blk == 0)
        def _init():
            z = jnp.zeros((b_blk, H), jnp.float32)
            h0_ref[...] = z
            c0_ref[...] = z
            h1_ref[...] = z
            c1_ref[...] = z
            rec0_ref[...] = jnp.zeros((b_blk, 4 * H), jnp.float32)

        g0_ref[...] = jnp.dot(x_ref[0], w_ih0_ref[...],
                              preferred_element_type=jnp.float32)

        w_step = w_step_ref[...]
        b1b = jnp.broadcast_to(b1_ref[...], (b_blk, 4 * H))

        def lstm_block(n_steps):
            def step(t, carry):
                h0, c0, h1, c1, rec0 = carry
                row = pl.multiple_of(t * b_blk, b_blk)
                g0 = g0_ref[pl.ds(row, b_blk), :] + rec0
                s0 = jax.nn.sigmoid(g0[:, :3 * H])
                gg0 = jnp.tanh(g0[:, 3 * H:])
                c0n = s0[:, H:2 * H] * c0 + s0[:, :H] * gg0
                h0n = s0[:, 2 * H:] * jnp.tanh(c0n)

                lhs = jnp.concatenate([h0n, h1], axis=-1).astype(w_step.dtype)
                big = jnp.dot(lhs, w_step, preferred_element_type=jnp.float32)
                g1 = big[:, :4 * H] + b1b
                rec0n = big[:, 4 * H:]

                s1 = jax.nn.sigmoid(g1[:, :3 * H])
                gg1 = jnp.tanh(g1[:, 3 * H:])
                c1n = s1[:, H:2 * H] * c1 + s1[:, :H] * gg1
                h1n = s1[:, 2 * H:] * jnp.tanh(c1n)
                return h0n, c0n, h1n, c1n, rec0n

            carry = (h0_ref[...], c0_ref[...], h1_ref[...], c1_ref[...],
                     rec0_ref[...])
            h0, c0, h1, c1, rec0 = jax.lax.fori_loop(
                0, n_steps, step, carry, unroll=min(unroll, n_steps))
            h0_ref[...] = h0
            c0_ref[...] = c0
            h1_ref[...] = h1
            c1_ref[...] = c1
            rec0_ref[...] = rec0

        if rem == block_t:
            lstm_block(block_t)
        else:
            @pl.when(t_blk < n_t - 1)
            def _full():
                lstm_block(block_t)

            @pl.when(t_blk == n_t - 1)
            def _tail():
                lstm_block(rem)

        @pl.when(t_blk == n_t - 1)
        def _epilogue():
            h = jnp.maximum(
                jnp.dot(h1_ref[...], w_d1_ref[...],
                        preferred_element_type=jnp.float32) + b_d1_ref[...],
                0.0)
            h = jnp.maximum(
                jnp.dot(h, w_d2_ref[...],
                        preferred_element_type=jnp.float32) + b_d2_ref[...],
                0.0)
            out_ref[0] = (
                jnp.dot(h, w_head_ref[...], preferred_element_type=jnp.float32)
                + b_head_ref[...])

    return body


def kernel(beatmap_features, positions, w_ih0, w_step, b1, w_d1, b_d1,
           w_d2, b_d2, w_head, b_head, *, block_t=128):
    H = HIDDEN
    x = jnp.concatenate([beatmap_features, positions],
                        axis=-1).astype(jnp.float32)
    B, T, C = x.shape

    b_pad = _round_up(max(B, 1), 8)
    n_b = 2 if b_pad % 16 == 0 else 1
    b_blk = b_pad // n_b

    bt = _round_up(max(8, min(block_t, _round_up(T, 8))), 8)
    T_pad = _round_up(T, bt)
    n_t = T_pad // bt

    x = jnp.transpose(x, (1, 0, 2))
    x = jnp.pad(x, ((0, T_pad - T), (0, b_pad - B), (0, 0)))
    x = jnp.concatenate(
        [x, jnp.ones((T_pad, b_pad, 1), jnp.float32)], axis=-1)
    x3 = x.reshape(T_pad, n_b, b_blk, C + 1).transpose(1, 0, 2, 3)
    x3 = x3.reshape(n_b, T_pad * b_blk, C + 1)

    L2 = w_head.shape[1]
    const = lambda b, t: (0, 0)

    out = pl.pallas_call(
        _make_body(seq_len=T, block_t=bt, n_t=n_t, b_blk=b_blk),
        grid=(n_b, n_t),
        in_specs=[
            pl.BlockSpec((1, bt * b_blk, C + 1), lambda b, t: (b, t, 0)),
            pl.BlockSpec(w_ih0.shape, const),
            pl.BlockSpec(w_step.shape, const),
            pl.BlockSpec(b1.shape, const),
            pl.BlockSpec(w_d1.shape, const),
            pl.BlockSpec(b_d1.shape, const),
            pl.BlockSpec(w_d2.shape, const),
            pl.BlockSpec(b_d2.shape, const),
            pl.BlockSpec(w_head.shape, const),
            pl.BlockSpec(b_head.shape, const),
        ],
        out_specs=pl.BlockSpec((1, b_blk, L2), lambda b, t: (b, 0, 0)),
        out_shape=jax.ShapeDtypeStruct((n_b, b_blk, L2), jnp.float32),
        scratch_shapes=[
            pltpu.VMEM((b_blk, H), jnp.float32),
            pltpu.VMEM((b_blk, H), jnp.float32),
            pltpu.VMEM((b_blk, H), jnp.float32),
            pltpu.VMEM((b_blk, H), jnp.float32),
            pltpu.VMEM((b_blk, 4 * H), jnp.float32),
            pltpu.VMEM((bt * b_blk, 4 * H), jnp.float32),
        ],
        compiler_params=pltpu.CompilerParams(
            dimension_semantics=("parallel", "arbitrary"),
        ),
    )(x3, w_ih0, w_step, b1, w_d1, b_d1, w_d2, b_d2, w_head, b_head)

    out = out.reshape(n_b * b_blk, L2)
    L = L2 // 2
    return out[:B, :L], out[:B, L:]
